# baseline (speedup 1.0000x reference)
import functools

import jax
import jax.numpy as jnp
from jax import lax
from jax.experimental import pallas as pl
from jax.experimental.pallas import tpu as pltpu
from jax.experimental.pallas import tpu_sc as plsc

EMB_D = 300
EMB_D_PAD = 304
SEQ_L = 20
N_SEQ = 26624
N_WORKERS = 32
SEQ_PER_W = N_SEQ // N_WORKERS
G = 8
GT = G * SEQ_L
NG = SEQ_PER_W // G
N_CHUNK = 19

_CHUNK_OFF = tuple(min(16 * j, EMB_D - 16) for j in range(N_CHUNK))


def _emb_body(x_hbm, w_hbm, out_hbm, idx_all, rows0, rows1, out0, out1,
              sem0, sem1, osem0, osem1):
    wid = lax.axis_index("c") * 16 + lax.axis_index("s")
    seq_base = wid * SEQ_PER_W
    lane = lax.broadcasted_iota(jnp.int32, (16,), 0)

    def gather_descs(g, rows_v, sem):
        return (
            pltpu.make_async_copy(
                w_hbm.at[idx_all.at[pl.ds(g * GT, 80)]],
                rows_v.at[pl.ds(0, 80)], sem),
            pltpu.make_async_copy(
                w_hbm.at[idx_all.at[pl.ds(g * GT + 80, 80)]],
                rows_v.at[pl.ds(80, 80)], sem),
        )

    def start_gather(g, rows_v, sem):
        for d in gather_descs(g, rows_v, sem):
            d.start()

    def wait_gather(g, rows_v, sem):
        for d in gather_descs(g, rows_v, sem):
            d.wait()

    def out_desc(g, out_v, osem):
        return pltpu.make_async_copy(
            out_v, out_hbm.at[pl.ds((seq_base + g * G) * EMB_D, G * EMB_D)],
            osem)

    pltpu.sync_copy(x_hbm.at[pl.ds(seq_base * SEQ_L, SEQ_PER_W * SEQ_L)],
                    idx_all)
    start_gather(0, rows0, sem0)
    start_gather(1, rows1, sem1)

    def seq_body(s, g, rows_v, out_v):
        t0 = g * GT + s * SEQ_L
        c1 = idx_all[pl.ds(t0, 16)]
        c2 = idx_all[pl.ds(t0 + 4, 16)]
        one = jnp.full((16,), 1.0, jnp.float32)
        zero = jnp.full((16,), 0.0, jnp.float32)
        m1 = jnp.where(c1 != 0, one, zero)
        m2 = jnp.where((c2 != 0) & (lane >= 12), one, zero)
        cnt = m1 + m2
        for sh in (1, 2, 4, 8):
            cnt = cnt + cnt.at[lane ^ sh].get(mode="promise_in_bounds")
        inv = one / jnp.maximum(cnt, one)
        r0 = s * SEQ_L
        o0 = s * EMB_D
        for j in range(N_CHUNK):
            off = _CHUNK_OFF[j]
            vals = [rows_v[r0 + l, pl.ds(off, 16)] for l in range(SEQ_L)]
            while len(vals) > 1:
                vals = [vals[k] + vals[k + 1] for k in range(0, len(vals) - 1, 2)] + (
                    [vals[-1]] if len(vals) % 2 else [])
            out_v[pl.ds(o0 + off, 16)] = vals[0] * inv

    def pair_body(i, carry):
        for p in (0, 1):
            g = 2 * i + p
            rows_v = rows0 if p == 0 else rows1
            sem = sem0 if p == 0 else sem1
            out_v = out0 if p == 0 else out1
            osem = osem0 if p == 0 else osem1
            wait_gather(g, rows_v, sem)

            @pl.when(g >= 2)
            def _():
                out_desc(g - 2, out_v, osem).wait()

            plsc.parallel_loop(0, G, 1, unroll=2)(
                functools.partial(seq_body, g=g, rows_v=rows_v, out_v=out_v))
            out_desc(g, out_v, osem).start()
            nxt = g + 2

            @pl.when(nxt < NG)
            def _():
                start_gather(nxt, rows_v, sem)
        return carry

    lax.fori_loop(0, NG // 2, pair_body, 0)
    out_desc(NG - 2, out0, osem0).wait()
    out_desc(NG - 1, out1, osem1).wait()


_emb = functools.partial(
    pl.kernel,
    out_type=jax.ShapeDtypeStruct((N_SEQ * EMB_D,), jnp.float32),
    mesh=plsc.VectorSubcoreMesh(core_axis_name="c", subcore_axis_name="s"),
    scratch_types=[
        pltpu.VMEM((SEQ_PER_W * SEQ_L,), jnp.int32),
        pltpu.VMEM((GT, EMB_D_PAD), jnp.float32),
        pltpu.VMEM((GT, EMB_D_PAD), jnp.float32),
        pltpu.VMEM((G * EMB_D,), jnp.float32),
        pltpu.VMEM((G * EMB_D,), jnp.float32),
        pltpu.SemaphoreType.DMA,
        pltpu.SemaphoreType.DMA,
        pltpu.SemaphoreType.DMA,
        pltpu.SemaphoreType.DMA,
    ],
    compiler_params=pltpu.CompilerParams(use_tc_tiling_on_sc=False),
)(_emb_body)


def kernel(x, W):
    b, nk, _ = x.shape
    w_pad = jnp.pad(W, ((0, 0), (0, EMB_D_PAD - EMB_D)))
    pooled = _emb(x.reshape(-1), w_pad)
    return pooled.reshape(b, nk, EMB_D)


# speedup vs baseline: 1.4001x; 1.4001x over previous
import functools

import jax
import jax.numpy as jnp
from jax import lax
from jax.experimental import pallas as pl
from jax.experimental.pallas import tpu as pltpu
from jax.experimental.pallas import tpu_sc as plsc

EMB_D = 300
EMB_D_PAD = 304
SEQ_L = 20
N_SEQ = 26624
N_WORKERS = 32
SEQ_PER_W = N_SEQ // N_WORKERS
G = 8
GT = G * SEQ_L
NG = SEQ_PER_W // G
N_CHUNK = 19

_CHUNK_OFF = tuple(min(16 * j, EMB_D - 16) for j in range(N_CHUNK))


def _emb_body(x_hbm, w_hbm, out_hbm, idx_all, rows0, rows1, out0, out1,
              sem0, sem1, osem0, osem1):
    wid = lax.axis_index("c") * 16 + lax.axis_index("s")
    seq_base = wid * SEQ_PER_W
    lane = lax.broadcasted_iota(jnp.int32, (16,), 0)

    def gather_descs(g, rows_v, sem):
        return (
            pltpu.make_async_copy(
                w_hbm.at[idx_all.at[pl.ds(g * GT, 80)]],
                rows_v.at[pl.ds(0, 80)], sem),
            pltpu.make_async_copy(
                w_hbm.at[idx_all.at[pl.ds(g * GT + 80, 80)]],
                rows_v.at[pl.ds(80, 80)], sem),
        )

    def start_gather(g, rows_v, sem):
        for d in gather_descs(g, rows_v, sem):
            d.start()

    def wait_gather(g, rows_v, sem):
        for d in gather_descs(g, rows_v, sem):
            d.wait()

    def out_desc(g, out_v, osem):
        return pltpu.make_async_copy(
            out_v, out_hbm.at[pl.ds((seq_base + g * G) * EMB_D, G * EMB_D)],
            osem)

    pltpu.sync_copy(x_hbm.at[pl.ds(seq_base * SEQ_L, SEQ_PER_W * SEQ_L)],
                    idx_all)
    start_gather(0, rows0, sem0)
    start_gather(1, rows1, sem1)

    def seq_body(s, carry, g, rows_v, out_v):
        t0 = g * GT + s * SEQ_L
        c1 = idx_all[pl.ds(t0, 16)]
        c2 = idx_all[pl.ds(t0 + 4, 16)]
        one = jnp.full((16,), 1.0, jnp.float32)
        zero = jnp.full((16,), 0.0, jnp.float32)
        m1 = jnp.where(c1 != 0, one, zero)
        m2 = jnp.where((c2 != 0) & (lane >= 12), one, zero)
        cnt = m1 + m2
        for sh in (1, 2, 4, 8):
            cnt = cnt + cnt.at[lane ^ sh].get(mode="promise_in_bounds")
        inv = one / jnp.maximum(cnt, one)
        r0 = s * SEQ_L
        o0 = s * EMB_D
        for j in range(N_CHUNK):
            off = _CHUNK_OFF[j]
            vals = [rows_v[r0 + l, pl.ds(off, 16)] for l in range(SEQ_L)]
            while len(vals) > 1:
                vals = [vals[k] + vals[k + 1] for k in range(0, len(vals) - 1, 2)] + (
                    [vals[-1]] if len(vals) % 2 else [])
            out_v[pl.ds(o0 + off, 16)] = vals[0] * inv
        return carry

    def pair_body(i, carry):
        for p in (0, 1):
            g = 2 * i + p
            rows_v = rows0 if p == 0 else rows1
            sem = sem0 if p == 0 else sem1
            out_v = out0 if p == 0 else out1
            osem = osem0 if p == 0 else osem1
            wait_gather(g, rows_v, sem)

            @pl.when(g >= 2)
            def _():
                out_desc(g - 2, out_v, osem).wait()

            lax.fori_loop(0, G,
                          functools.partial(seq_body, g=g, rows_v=rows_v,
                                            out_v=out_v), 0)
            out_desc(g, out_v, osem).start()
            nxt = g + 2

            @pl.when(nxt < NG)
            def _():
                start_gather(nxt, rows_v, sem)
        return carry

    lax.fori_loop(0, NG // 2, pair_body, 0)
    out_desc(NG - 2, out0, osem0).wait()
    out_desc(NG - 1, out1, osem1).wait()


_emb = functools.partial(
    pl.kernel,
    out_type=jax.ShapeDtypeStruct((N_SEQ * EMB_D,), jnp.float32),
    mesh=plsc.VectorSubcoreMesh(core_axis_name="c", subcore_axis_name="s"),
    scratch_types=[
        pltpu.VMEM((SEQ_PER_W * SEQ_L,), jnp.int32),
        pltpu.VMEM((GT, EMB_D_PAD), jnp.float32),
        pltpu.VMEM((GT, EMB_D_PAD), jnp.float32),
        pltpu.VMEM((G * EMB_D,), jnp.float32),
        pltpu.VMEM((G * EMB_D,), jnp.float32),
        pltpu.SemaphoreType.DMA,
        pltpu.SemaphoreType.DMA,
        pltpu.SemaphoreType.DMA,
        pltpu.SemaphoreType.DMA,
    ],
    compiler_params=pltpu.CompilerParams(use_tc_tiling_on_sc=False),
)(_emb_body)


def _pad_body(w_ref, o_ref):
    o_ref[:, :EMB_D] = w_ref[...]
    o_ref[:, EMB_D:] = jnp.zeros((_PAD_R, EMB_D_PAD - EMB_D), jnp.float32)


_PAD_R = 1000

_pad_tc = pl.pallas_call(
    _pad_body,
    out_shape=jax.ShapeDtypeStruct((100000, EMB_D_PAD), jnp.float32),
    grid=(100000 // _PAD_R,),
    in_specs=[pl.BlockSpec((_PAD_R, EMB_D), lambda i: (i, 0))],
    out_specs=pl.BlockSpec((_PAD_R, EMB_D_PAD), lambda i: (i, 0)),
)


def kernel(x, W):
    b, nk, _ = x.shape
    w_pad = _pad_tc(W)
    pooled = _emb(x.reshape(-1), w_pad)
    return pooled.reshape(b, nk, EMB_D)
